# Initial kernel scaffold; baseline (speedup 1.0000x reference)
import jax
import jax.numpy as jnp
from jax.experimental import pallas as pl


def kernel(x, hour_table, day_table, month_table):
    raise NotImplementedError("write your pallas kernel here")


# speedup vs baseline: 1.6535x; 1.6535x over previous
import functools

import jax
import jax.numpy as jnp
from jax import lax
from jax.experimental import pallas as pl
from jax.experimental.pallas import tpu as pltpu
from jax.experimental.pallas import tpu_sc as plsc

TIME_DIM = 64
HOUR_SIZE = 24
DAY_SIZE = 32
MONTH_SIZE = 13
BATCH = 16384

NC = 2
NS = 16
L = 16
NW = NC * NS
B_PER_W = BATCH // NW
N_CHUNKS = B_PER_W // L
IDX_CHUNK = 128
N_IDX = B_PER_W // IDX_CHUNK


def _body(x_hbm, hour_hbm, day_hbm, month_hbm, out_hbm,
          xv, hidx, didx, midx, hbuf, dbuf, mbuf, sem):
    wid = lax.axis_index("s") * NC + lax.axis_index("c")
    base = wid * B_PER_W

    pltpu.sync_copy(x_hbm.at[pl.ds(base * 3, B_PER_W * 3)], xv)

    lane3 = lax.iota(jnp.int32, L) * 3
    for c in range(N_CHUNKS):
        b0 = c * L * 3
        vm = plsc.load_gather(xv, [lane3 + b0])
        vd = plsc.load_gather(xv, [lane3 + (b0 + 1)])
        vh = plsc.load_gather(xv, [lane3 + (b0 + 2)])
        ih = (vh * HOUR_SIZE).astype(jnp.int32)
        id_ = (vd * DAY_SIZE).astype(jnp.int32)
        im = (vm * MONTH_SIZE).astype(jnp.int32)
        r, cc = divmod(c, IDX_CHUNK // L)
        sl = pl.ds(cc * L, L)
        hidx[r, sl] = ih
        didx[r, sl] = id_
        midx[r, sl] = im

    descs = []
    for j in range(N_IDX):
        dst = pl.ds(j * IDX_CHUNK, IDX_CHUNK)
        descs.append(pltpu.async_copy(hour_hbm.at[hidx.at[j]], hbuf.at[dst], sem))
        descs.append(pltpu.async_copy(day_hbm.at[didx.at[j]], dbuf.at[dst], sem))
        descs.append(pltpu.async_copy(month_hbm.at[midx.at[j]], mbuf.at[dst], sem))
    for dsc in descs:
        dsc.wait()

    @pl.loop(0, B_PER_W, unroll=4)
    def _add(r):
        for cc in range(TIME_DIM // L):
            sl = pl.ds(cc * L, L)
            hbuf[r, sl] = hbuf[r, sl] + dbuf[r, sl] + mbuf[r, sl]

    pltpu.sync_copy(hbuf, out_hbm.at[pl.ds(base, B_PER_W)])


@jax.jit
def kernel(x, hour_table, day_table, month_table):
    run = pl.kernel(
        _body,
        out_type=jax.ShapeDtypeStruct((BATCH, TIME_DIM), jnp.float32),
        mesh=plsc.VectorSubcoreMesh(
            core_axis_name="c", subcore_axis_name="s",
            num_cores=NC, num_subcores=NS),
        scratch_types=[
            pltpu.VMEM((B_PER_W * 3,), jnp.float32),
            pltpu.VMEM((N_IDX, IDX_CHUNK), jnp.int32),
            pltpu.VMEM((N_IDX, IDX_CHUNK), jnp.int32),
            pltpu.VMEM((N_IDX, IDX_CHUNK), jnp.int32),
            pltpu.VMEM((B_PER_W, TIME_DIM), jnp.float32),
            pltpu.VMEM((B_PER_W, TIME_DIM), jnp.float32),
            pltpu.VMEM((B_PER_W, TIME_DIM), jnp.float32),
            pltpu.SemaphoreType.DMA,
        ],
        compiler_params=pltpu.CompilerParams(
            needs_layout_passes=False, use_tc_tiling_on_sc=False),
    )
    return run(x.reshape(-1), hour_table, day_table, month_table)


# speedup vs baseline: 1.9696x; 1.1912x over previous
import jax
import jax.numpy as jnp
from jax import lax
from jax.experimental import pallas as pl
from jax.experimental.pallas import tpu as pltpu
from jax.experimental.pallas import tpu_sc as plsc

TIME_DIM = 64
HOUR_SIZE = 24
DAY_SIZE = 32
MONTH_SIZE = 13
BATCH = 16384

NC = 2
NS = 16
L = 16
NW = NC * NS
B_PER_W = BATCH // NW
N_CHUNKS = B_PER_W // L

HOUR_OFF = 0
DAY_OFF = HOUR_SIZE * TIME_DIM
MONTH_OFF = DAY_OFF + DAY_SIZE * TIME_DIM
TABLE_WORDS = MONTH_OFF + MONTH_SIZE * TIME_DIM


def _body(x_hbm, hour_hbm, day_hbm, month_hbm, out_hbm, xv, tv, ov, sem):
    wid = lax.axis_index("s") * NC + lax.axis_index("c")
    base = wid * B_PER_W

    cp_x = pltpu.async_copy(x_hbm.at[pl.ds(base * 3, B_PER_W * 3)], xv, sem)
    cp_h = pltpu.async_copy(hour_hbm, tv.at[pl.ds(HOUR_OFF, DAY_OFF)], sem)
    cp_d = pltpu.async_copy(day_hbm, tv.at[pl.ds(DAY_OFF, MONTH_OFF - DAY_OFF)], sem)
    cp_m = pltpu.async_copy(
        month_hbm, tv.at[pl.ds(MONTH_OFF, TABLE_WORDS - MONTH_OFF)], sem)
    for cp in (cp_x, cp_h, cp_d, cp_m):
        cp.wait()

    lane = lax.iota(jnp.int32, L)
    lane3 = lane * 3
    lane64 = lane * TIME_DIM

    @pl.loop(0, N_CHUNKS)
    def _chunk(c):
        b0 = c * (L * 3)
        vm = plsc.load_gather(xv, [lane3 + b0])
        vd = plsc.load_gather(xv, [lane3 + (b0 + 1)])
        vh = plsc.load_gather(xv, [lane3 + (b0 + 2)])
        ih = (vh * HOUR_SIZE).astype(jnp.int32) * TIME_DIM + HOUR_OFF
        id_ = (vd * DAY_SIZE).astype(jnp.int32) * TIME_DIM + DAY_OFF
        im = (vm * MONTH_SIZE).astype(jnp.int32) * TIME_DIM + MONTH_OFF
        ob = lane64 + c * (L * TIME_DIM)

        @plsc.parallel_loop(0, TIME_DIM, unroll=8)
        def _col(j):
            a = plsc.load_gather(tv, [ih + j])
            b = plsc.load_gather(tv, [id_ + j])
            m = plsc.load_gather(tv, [im + j])
            plsc.store_scatter(ov, [ob + j], (a + b) + m)

    pltpu.sync_copy(ov, out_hbm.at[pl.ds(base * TIME_DIM, B_PER_W * TIME_DIM)])


@jax.jit
def kernel(x, hour_table, day_table, month_table):
    run = pl.kernel(
        _body,
        out_type=jax.ShapeDtypeStruct((BATCH * TIME_DIM,), jnp.float32),
        mesh=plsc.VectorSubcoreMesh(
            core_axis_name="c", subcore_axis_name="s",
            num_cores=NC, num_subcores=NS),
        scratch_types=[
            pltpu.VMEM((B_PER_W * 3,), jnp.float32),
            pltpu.VMEM((TABLE_WORDS,), jnp.float32),
            pltpu.VMEM((B_PER_W * TIME_DIM,), jnp.float32),
            pltpu.SemaphoreType.DMA,
        ],
        compiler_params=pltpu.CompilerParams(
            needs_layout_passes=False, use_tc_tiling_on_sc=False),
    )
    out = run(x.reshape(-1), hour_table.reshape(-1), day_table.reshape(-1),
              month_table.reshape(-1))
    return out.reshape(BATCH, TIME_DIM)


# speedup vs baseline: 3.2422x; 1.6461x over previous
import jax
import jax.numpy as jnp
from jax import lax
from jax.experimental import pallas as pl
from jax.experimental.pallas import tpu as pltpu
from jax.experimental.pallas import tpu_sc as plsc

TIME_DIM = 64
HOUR_SIZE = 24
DAY_SIZE = 32
MONTH_SIZE = 13
BATCH = 16384

NC = 2
NS = 16
L = 16
NW = NC * NS
B_PER_W = BATCH // NW
N_CHUNKS = B_PER_W // L

HOUR_OFF = 0
DAY_OFF = HOUR_SIZE * TIME_DIM
MONTH_OFF = DAY_OFF + DAY_SIZE * TIME_DIM
TABLE_WORDS = MONTH_OFF + MONTH_SIZE * TIME_DIM


def _body(x_hbm, hour_hbm, day_hbm, month_hbm, out_hbm,
          xv, tv, ov, ah, ad, am, sem):
    wid = lax.axis_index("s") * NC + lax.axis_index("c")
    base = wid * B_PER_W

    cp_x = pltpu.async_copy(x_hbm.at[pl.ds(base * 3, B_PER_W * 3)], xv, sem)
    cp_h = pltpu.async_copy(hour_hbm, tv.at[pl.ds(HOUR_OFF, DAY_OFF)], sem)
    cp_d = pltpu.async_copy(day_hbm, tv.at[pl.ds(DAY_OFF, MONTH_OFF - DAY_OFF)], sem)
    cp_m = pltpu.async_copy(
        month_hbm, tv.at[pl.ds(MONTH_OFF, TABLE_WORDS - MONTH_OFF)], sem)
    for cp in (cp_x, cp_h, cp_d, cp_m):
        cp.wait()

    lane = lax.iota(jnp.int32, L)
    lane3 = lane * 3

    @plsc.parallel_loop(0, N_CHUNKS, unroll=4)
    def _chunk(c):
        b0 = c * (L * 3)
        vm = plsc.load_gather(xv, [lane3 + b0])
        vd = plsc.load_gather(xv, [lane3 + (b0 + 1)])
        vh = plsc.load_gather(xv, [lane3 + (b0 + 2)])
        sl = pl.ds(c * L, L)
        ah[sl] = (vh * HOUR_SIZE).astype(jnp.int32) * TIME_DIM + HOUR_OFF
        ad[sl] = (vd * DAY_SIZE).astype(jnp.int32) * TIME_DIM + DAY_OFF
        am[sl] = (vm * MONTH_SIZE).astype(jnp.int32) * TIME_DIM + MONTH_OFF

    @plsc.parallel_loop(0, N_CHUNKS)
    def _lookup(c):
        sl = pl.ds(c * L, L)
        hv = ah[sl]
        dv = ad[sl]
        mv = am[sl]
        for r in range(L):
            ridx = jnp.full((L,), r, jnp.int32)
            hb = jnp.take_along_axis(hv, ridx, axis=0,
                                     mode="promise_in_bounds") + lane
            db = jnp.take_along_axis(dv, ridx, axis=0,
                                     mode="promise_in_bounds") + lane
            mb = jnp.take_along_axis(mv, ridx, axis=0,
                                     mode="promise_in_bounds") + lane
            ob = (c * L + r) * TIME_DIM
            for g in range(TIME_DIM // L):
                va = plsc.load_gather(tv, [hb + g * L])
                vb = plsc.load_gather(tv, [db + g * L])
                vc = plsc.load_gather(tv, [mb + g * L])
                ov[pl.ds(ob + g * L, L)] = (va + vb) + vc

    pltpu.sync_copy(ov, out_hbm.at[pl.ds(base * TIME_DIM, B_PER_W * TIME_DIM)])


@jax.jit
def kernel(x, hour_table, day_table, month_table):
    run = pl.kernel(
        _body,
        out_type=jax.ShapeDtypeStruct((BATCH * TIME_DIM,), jnp.float32),
        mesh=plsc.VectorSubcoreMesh(
            core_axis_name="c", subcore_axis_name="s",
            num_cores=NC, num_subcores=NS),
        scratch_types=[
            pltpu.VMEM((B_PER_W * 3,), jnp.float32),
            pltpu.VMEM((TABLE_WORDS,), jnp.float32),
            pltpu.VMEM((B_PER_W * TIME_DIM,), jnp.float32),
            pltpu.VMEM((B_PER_W,), jnp.int32),
            pltpu.VMEM((B_PER_W,), jnp.int32),
            pltpu.VMEM((B_PER_W,), jnp.int32),
            pltpu.SemaphoreType.DMA,
        ],
        compiler_params=pltpu.CompilerParams(
            needs_layout_passes=False, use_tc_tiling_on_sc=False),
    )
    out = run(x.reshape(-1), hour_table.reshape(-1), day_table.reshape(-1),
              month_table.reshape(-1))
    return out.reshape(BATCH, TIME_DIM)
